# baseline (speedup 1.0000x reference)
import functools

import jax
import jax.numpy as jnp
from jax import lax
from jax.experimental import pallas as pl
from jax.experimental.pallas import tpu as pltpu
from jax.experimental.pallas import tpu_sc as plsc

N = 100000
VOCAB = 1000
D = 128
POS_D = 3
OUT_D = D + POS_D

NC, NS = 2, 16
NW = NC * NS
SB = 128
GK = 5

SPLIT = 50000
CONCAT_ROWS = 10000


def _make_gather_body(start, nrows, iters):
    chunk = iters * SB
    last = nrows - chunk

    def body(species_hbm, table_hbm, emb_hbm, idx_v, rows_v, tbl_s, isem,
             gsem, wsem):
        wid = lax.axis_index("s") * NC + lax.axis_index("c")
        local = jnp.minimum(wid * chunk, last)
        local = pl.multiple_of(local, 8)

        @pl.when(lax.axis_index("s") == 0)
        def _stage_table():
            pltpu.sync_copy(table_hbm, tbl_s)

        plsc.subcore_barrier()

        for j in range(iters):
            sub = pl.multiple_of(start + local + j * SB, 8)
            pltpu.async_copy(species_hbm.at[pl.ds(sub, SB)], idx_v.at[j],
                             isem)
        for _ in range(iters):
            pltpu.make_async_copy(species_hbm.at[pl.ds(0, SB)], idx_v.at[0],
                                  isem).wait()

        def do_group(jj0, count):
            for b in range(count):
                pltpu.async_copy(tbl_s.at[idx_v.at[jj0 + b]],
                                 rows_v.at[b], gsem)
            for b in range(count):
                pltpu.make_async_copy(tbl_s.at[idx_v.at[0]],
                                      rows_v.at[b], gsem).wait()
            for b in range(count):
                sub = pl.multiple_of(local + (jj0 + b) * SB, 8)
                pltpu.async_copy(rows_v.at[b], emb_hbm.at[pl.ds(sub, SB)],
                                 wsem)
            for b in range(count):
                pltpu.make_async_copy(rows_v.at[b],
                                      emb_hbm.at[pl.ds(0, SB)], wsem).wait()

        full, tail = divmod(iters, GK)

        def group(t, carry):
            do_group(t * GK, GK)
            return carry

        lax.fori_loop(0, full, group, 0)
        if tail:
            do_group(full * GK, tail)

    return body


@functools.lru_cache(maxsize=None)
def _build_gather(start, nrows):
    iters = -(-nrows // (NW * SB))
    mesh = plsc.VectorSubcoreMesh(core_axis_name="c", subcore_axis_name="s")
    return pl.kernel(
        _make_gather_body(start, nrows, iters),
        out_type=jax.ShapeDtypeStruct((nrows, D), jnp.float32),
        mesh=mesh,
        compiler_params=pltpu.CompilerParams(use_tc_tiling_on_sc=False),
        scratch_types=[
            pltpu.VMEM((iters, SB), jnp.int32),
            pltpu.VMEM((GK, SB, D), jnp.float32),
            pltpu.VMEM_SHARED((VOCAB, D), jnp.float32),
            pltpu.SemaphoreType.DMA,
            pltpu.SemaphoreType.DMA,
            pltpu.SemaphoreType.DMA,
        ],
    )


def _concat_first_body(emb_ref, pos_ref, out_ref):
    out_ref[:, pl.ds(0, D)] = emb_ref[...]
    out_ref[:, pl.ds(D, POS_D)] = pos_ref[...]


def _concat_second_body(emb_ref, pos_ref, alias_ref, out_ref):
    del alias_ref
    out_ref[:, pl.ds(0, D)] = emb_ref[...]
    out_ref[:, pl.ds(D, POS_D)] = pos_ref[...]


@functools.lru_cache(maxsize=None)
def _build_concat(block_offset, nrows, aliased):
    grid = (nrows // CONCAT_ROWS,)
    emb_spec = pl.BlockSpec((CONCAT_ROWS, D), lambda i: (i, 0))
    pos_spec = pl.BlockSpec((CONCAT_ROWS, POS_D),
                            lambda i: (i + block_offset, 0))
    out_spec = pl.BlockSpec((CONCAT_ROWS, OUT_D),
                            lambda i: (i + block_offset, 0))
    if aliased:
        in_specs = [emb_spec, pos_spec,
                    pl.BlockSpec(memory_space=pltpu.MemorySpace.HBM)]
        body = _concat_second_body
        aliases = {2: 0}
    else:
        in_specs = [emb_spec, pos_spec]
        body = _concat_first_body
        aliases = {}
    return pl.pallas_call(
        body,
        grid=grid,
        in_specs=in_specs,
        out_specs=out_spec,
        out_shape=jax.ShapeDtypeStruct((N, OUT_D), jnp.float32),
        input_output_aliases=aliases,
    )


@jax.jit
def kernel(species, positions, embedding_table):
    s = species.astype(jnp.int32)
    emb = _build_gather(0, N)(s, embedding_table)
    return _build_concat(0, N, False)(emb, positions)


# speedup vs baseline: 1.0176x; 1.0176x over previous
import functools

import jax
import jax.numpy as jnp
from jax import lax
from jax.experimental import pallas as pl
from jax.experimental.pallas import tpu as pltpu
from jax.experimental.pallas import tpu_sc as plsc

N = 100000
VOCAB = 1000
D = 128
POS_D = 3
OUT_D = D + POS_D

NC, NS = 2, 16
NW = NC * NS
SB = 128
GK = 5

SPLIT = 50000
CONCAT_ROWS = 10000


def _make_gather_body(start, nrows, iters):
    chunk = iters * SB
    last = nrows - chunk

    def body(species_hbm, table_hbm, emb_hbm, idx_v, rows_v, tbl_s, isem,
             gsem, wsem):
        wid = lax.axis_index("s") * NC + lax.axis_index("c")
        local = jnp.minimum(wid * chunk, last)
        local = pl.multiple_of(local, 8)

        @pl.when(lax.axis_index("s") == 0)
        def _stage_table():
            pltpu.sync_copy(table_hbm, tbl_s)

        plsc.subcore_barrier()

        for j in range(iters):
            sub = pl.multiple_of(start + local + j * SB, 8)
            pltpu.async_copy(species_hbm.at[pl.ds(sub, SB)], idx_v.at[j],
                             isem)
        for _ in range(iters):
            pltpu.make_async_copy(species_hbm.at[pl.ds(0, SB)], idx_v.at[0],
                                  isem).wait()

        def do_group(jj0, count):
            for b in range(count):
                pltpu.async_copy(tbl_s.at[idx_v.at[jj0 + b]],
                                 rows_v.at[b], gsem)
            for b in range(count):
                pltpu.make_async_copy(tbl_s.at[idx_v.at[0]],
                                      rows_v.at[b], gsem).wait()
            for b in range(count):
                sub = pl.multiple_of(local + (jj0 + b) * SB, 8)
                pltpu.async_copy(rows_v.at[b], emb_hbm.at[pl.ds(sub, SB)],
                                 wsem)
            for b in range(count):
                pltpu.make_async_copy(rows_v.at[b],
                                      emb_hbm.at[pl.ds(0, SB)], wsem).wait()

        full, tail = divmod(iters, GK)

        def group(t, carry):
            do_group(t * GK, GK)
            return carry

        lax.fori_loop(0, full, group, 0)
        if tail:
            do_group(full * GK, tail)

    return body


@functools.lru_cache(maxsize=None)
def _build_gather(start, nrows):
    iters = -(-nrows // (NW * SB))
    mesh = plsc.VectorSubcoreMesh(core_axis_name="c", subcore_axis_name="s")
    return pl.kernel(
        _make_gather_body(start, nrows, iters),
        out_type=jax.ShapeDtypeStruct((nrows, D), jnp.float32),
        mesh=mesh,
        compiler_params=pltpu.CompilerParams(use_tc_tiling_on_sc=False),
        scratch_types=[
            pltpu.VMEM((iters, SB), jnp.int32),
            pltpu.VMEM((GK, SB, D), jnp.float32),
            pltpu.VMEM_SHARED((VOCAB, D), jnp.float32),
            pltpu.SemaphoreType.DMA,
            pltpu.SemaphoreType.DMA,
            pltpu.SemaphoreType.DMA,
        ],
    )


def _concat_first_body(emb_ref, pos_ref, out_ref):
    out_ref[:, pl.ds(0, D)] = emb_ref[...]
    out_ref[:, pl.ds(D, POS_D)] = pos_ref[...]


def _concat_second_body(emb_ref, pos_ref, alias_ref, out_ref):
    del alias_ref
    out_ref[:, pl.ds(0, D)] = emb_ref[...]
    out_ref[:, pl.ds(D, POS_D)] = pos_ref[...]


@functools.lru_cache(maxsize=None)
def _build_concat(block_offset, nrows, aliased):
    grid = (nrows // CONCAT_ROWS,)
    emb_spec = pl.BlockSpec((CONCAT_ROWS, D), lambda i: (i, 0))
    pos_spec = pl.BlockSpec((CONCAT_ROWS, POS_D),
                            lambda i: (i + block_offset, 0))
    out_spec = pl.BlockSpec((CONCAT_ROWS, OUT_D),
                            lambda i: (i + block_offset, 0))
    if aliased:
        in_specs = [emb_spec, pos_spec,
                    pl.BlockSpec(memory_space=pltpu.MemorySpace.HBM)]
        body = _concat_second_body
        aliases = {2: 0}
    else:
        in_specs = [emb_spec, pos_spec]
        body = _concat_first_body
        aliases = {}
    return pl.pallas_call(
        body,
        grid=grid,
        in_specs=in_specs,
        out_specs=out_spec,
        out_shape=jax.ShapeDtypeStruct((N, OUT_D), jnp.float32),
        input_output_aliases=aliases,
    )


@jax.jit
def kernel(species, positions, embedding_table):
    s = species.astype(jnp.int32)
    emb1 = _build_gather(0, SPLIT)(s, embedding_table)
    emb2 = _build_gather(SPLIT, N - SPLIT)(s, embedding_table)
    out1 = _build_concat(0, SPLIT, False)(emb1, positions)
    return _build_concat(SPLIT // CONCAT_ROWS, N - SPLIT, True)(
        emb2, positions, out1)
